# baseline (speedup 1.0000x reference)
import functools

import jax
import jax.numpy as jnp
from jax import lax
from jax.experimental import pallas as pl
from jax.experimental.pallas import tpu as pltpu
from jax.experimental.pallas import tpu_sc as plsc

B = 4096
L = 50
D = 16
LP = 64
NC_PAD = 1024
NR = 5
NR_PAD = 16
NW = 32
ROWS = B // NW


def _tc_precompute(c2e_p, r2e_p, Wk, Wv, Wo, bv2, bo2):
    def body(c2e_ref, r2e_ref, wk_ref, wv_ref, wo_ref, bv_ref, bo_ref,
             skt_ref, rkt_ref, svr_ref):
        c2e = c2e_ref[...]
        r2e = r2e_ref[...]
        wk0 = wk_ref[0:D, :]
        wk1 = wk_ref[D:2 * D, :]
        wv0 = wv_ref[0:D, :]
        wv1 = wv_ref[D:2 * D, :]
        wo = wo_ref[...]
        scale = 0.25
        sk = jnp.dot(c2e, wk0, preferred_element_type=jnp.float32) * scale
        skt_ref[...] = sk.T
        rk = jnp.dot(r2e, wk1, preferred_element_type=jnp.float32) * scale
        rkt_ref[...] = rk.T
        wvo0 = jnp.dot(wv0, wo, preferred_element_type=jnp.float32)
        wvo1 = jnp.dot(wv1, wo, preferred_element_type=jnp.float32)
        cb = jnp.dot(bv_ref[...], wo, preferred_element_type=jnp.float32) + bo_ref[...]
        sv = jnp.dot(c2e, wvo0, preferred_element_type=jnp.float32)
        rvp = jnp.dot(r2e, wvo1, preferred_element_type=jnp.float32) + cb
        for r in range(NR):
            svr_ref[:, r * D:(r + 1) * D] = sv + rvp[r, :]

    return pl.pallas_call(
        body,
        out_shape=(
            jax.ShapeDtypeStruct((D, NC_PAD), jnp.float32),
            jax.ShapeDtypeStruct((D, NR_PAD), jnp.float32),
            jax.ShapeDtypeStruct((NC_PAD, NR * D), jnp.float32),
        ),
    )(c2e_p, r2e_p, Wk, Wv, Wo, bv2, bo2)


def _sc_attention(nodes_hi, nodes_lo, hcomb, v2e128, Wq, bq2, skt, rkt, svr):
    mesh = plsc.VectorSubcoreMesh(core_axis_name="c", subcore_axis_name="s")
    UNROLL = 2
    NGRP = ROWS // 16

    @functools.partial(
        pl.kernel,
        mesh=mesh,
        compiler_params=pltpu.CompilerParams(
            needs_layout_passes=False, use_tc_tiling_on_sc=False),
        out_type=jax.ShapeDtypeStruct((B, D), jnp.float32),
        scratch_types=[
            pltpu.VMEM((D, NC_PAD), jnp.float32),
            pltpu.VMEM((D, NR_PAD), jnp.float32),
            pltpu.VMEM((NC_PAD * NR, D), jnp.float32),
            pltpu.VMEM((D, D), jnp.float32),
            pltpu.VMEM((1, D), jnp.float32),
            pltpu.VMEM((ROWS,), jnp.int32),
            pltpu.VMEM((NGRP, 16), jnp.int32),
            pltpu.VMEM((ROWS, 128), jnp.float32),
            pltpu.VMEM((ROWS, D), jnp.float32),
            pltpu.VMEM((ROWS, LP), jnp.int32),
            pltpu.VMEM((UNROLL, D), jnp.float32),
            pltpu.VMEM((ROWS, D), jnp.float32),
            pltpu.SemaphoreType.DMA,
        ],
    )
    def k(nhi_hbm, nlo_hbm, hcomb_hbm, v2e_hbm, wq_hbm, bq_hbm,
          skt_hbm, rkt_hbm, svr_hbm, out_hbm,
          skt_v, rkt_v, svr_v, wq_v, bq_v, nhi_v, nlo_v, buf128, vcrep_v,
          hcomb_v, rkq_buf, outbuf, sem):
        wid = lax.axis_index("c") * 16 + lax.axis_index("s")
        base = wid * ROWS

        pltpu.sync_copy(skt_hbm, skt_v)
        pltpu.sync_copy(rkt_hbm, rkt_v)
        pltpu.sync_copy(svr_hbm, svr_v)
        pltpu.sync_copy(wq_hbm, wq_v)
        pltpu.sync_copy(bq_hbm, bq_v)
        pltpu.sync_copy(nhi_hbm.at[pl.ds(base, ROWS)], nhi_v)
        pltpu.sync_copy(nlo_hbm.at[pl.ds(wid * NGRP, NGRP)], nlo_v)
        pltpu.sync_copy(hcomb_hbm.at[pl.ds(base, ROWS)], hcomb_v)
        pltpu.async_copy(v2e_hbm.at[nhi_v], buf128, sem).wait()

        iota = lax.iota(jnp.int32, 16)
        lanemask_last = iota < (L - 3 * 16)
        neg = jnp.full((16,), -1e30, jnp.float32)
        nchunk = LP // 16

        for g in range(NGRP):
            rowids = jnp.full((16,), g * 16, jnp.int32) + iota
            lovec = nlo_v[g, :]
            for d in range(D):
                vals = plsc.load_gather(buf128, [rowids, lovec + d])
                plsc.store_scatter(vcrep_v, [rowids, jnp.full((16,), d, jnp.int32)], vals)

        def one_row(i, slot):
            vcvec = vcrep_v[i, :]
            qa = [bq_v[0, :], jnp.zeros((16,), jnp.float32),
                  jnp.zeros((16,), jnp.float32), jnp.zeros((16,), jnp.float32)]
            for d in range(D):
                qa[d % 4] = qa[d % 4] + vcvec[d] * wq_v[d, :]
            q = (qa[0] + qa[1]) + (qa[2] + qa[3])
            qs = [q[d] for d in range(D)]
            ra = [jnp.zeros((16,), jnp.float32) for _ in range(4)]
            for d in range(D):
                ra[d % 4] = ra[d % 4] + qs[d] * rkt_v[d, :]
            rkq_buf[slot, :] = (ra[0] + ra[1]) + (ra[2] + ra[3])
            slotv = jnp.full((16,), slot, jnp.int32)
            chunks, combs = [], []
            for t in range(nchunk):
                packed = hcomb_v[i, pl.ds(16 * t, 16)]
                cv = jnp.bitwise_and(packed, jnp.full((16,), NC_PAD - 1, jnp.int32))
                rr = jnp.right_shift(packed, jnp.full((16,), 10, jnp.int32))
                combs.append(cv * NR + rr)
                sa = [plsc.load_gather(rkq_buf, [slotv, rr]),
                      jnp.zeros((16,), jnp.float32),
                      jnp.zeros((16,), jnp.float32),
                      jnp.zeros((16,), jnp.float32)]
                for d in range(D):
                    dvec = jnp.full((16,), d, jnp.int32)
                    sa[d % 4] = sa[d % 4] + qs[d] * plsc.load_gather(
                        skt_v, [dvec, cv])
                chunks.append((sa[0] + sa[1]) + (sa[2] + sa[3]))
            chunks[3] = jnp.where(lanemask_last, chunks[3], neg)
            m = jnp.max(jnp.maximum(jnp.maximum(chunks[0], chunks[1]),
                                    jnp.maximum(chunks[2], chunks[3])))
            es = [jnp.exp(c - m) for c in chunks]
            total = jnp.sum((es[0] + es[1]) + (es[2] + es[3]))
            inv = jnp.full((16,), 1.0, jnp.float32) / jnp.broadcast_to(
                total, (16,))
            oa = [jnp.zeros((16,), jnp.float32) for _ in range(4)]
            for t in range(nchunk):
                at = es[t] * inv
                for j in range(16):
                    l = 16 * t + j
                    if l >= L:
                        break
                    c = jnp.broadcast_to(combs[t][j], (16,))
                    row = plsc.load_gather(svr_v, [c, iota])
                    oa[l % 4] = oa[l % 4] + at[j] * row
            out = (oa[0] + oa[1]) + (oa[2] + oa[3])
            plsc.store_scatter(outbuf, [jnp.broadcast_to(i, (16,)), iota], out)

        def row_body(ii, carry):
            for u in range(UNROLL):
                one_row(ii * UNROLL + u, u)
            return carry

        lax.fori_loop(0, ROWS // UNROLL, row_body, 0)
        pltpu.sync_copy(outbuf, out_hbm.at[pl.ds(base, ROWS)])

    return k(nodes_hi, nodes_lo, hcomb, v2e128, Wq, bq2, skt, rkt, svr)


def kernel(nodes, history_vc, history_r, c2e_weight, r2e_weight, v2e_weight,
           Wq, bq, Wk, bk, Wv, bv, Wo, bo):
    nodes = nodes.astype(jnp.int32)
    nodes_hi = nodes >> 3
    nodes_lo = (nodes & 7) << 4
    nodes_lo = nodes_lo.reshape(B // 16, 16)
    hcomb = (history_r.astype(jnp.int32) << 10) | history_vc.astype(jnp.int32)
    hcomb = jnp.pad(hcomb, ((0, 0), (0, LP - L)))
    v2e128 = v2e_weight.reshape(-1, 128)
    c2e_p = jnp.pad(c2e_weight, ((0, NC_PAD - c2e_weight.shape[0]), (0, 0)))
    r2e_p = jnp.pad(r2e_weight, ((0, NR_PAD - r2e_weight.shape[0]), (0, 0)))
    bv2 = bv.reshape(1, D)
    bo2 = bo.reshape(1, D)
    bq2 = bq.reshape(1, D)
    skt, rkt, svr80 = _tc_precompute(c2e_p, r2e_p, Wk, Wv, Wo, bv2, bo2)
    svr = svr80.reshape(NC_PAD * NR, D)
    return _sc_attention(nodes_hi, nodes_lo, hcomb, v2e128, Wq, bq2,
                         skt, rkt, svr)


# speedup vs baseline: 5.5253x; 5.5253x over previous
import functools

import jax
import jax.numpy as jnp
from jax import lax
from jax.experimental import pallas as pl
from jax.experimental.pallas import tpu as pltpu
from jax.experimental.pallas import tpu_sc as plsc

B = 4096
L = 50
D = 16
LP = 64
NC_PAD = 1024
NR = 5
NR_PAD = 16
NW = 32
ROWS = B // NW


def _tc_precompute(c2e_p, r2e_p, Wk, Wv, Wo, bv2, bo2):
    def body(c2e_ref, r2e_ref, wk_ref, wv_ref, wo_ref, bv_ref, bo_ref,
             skt_ref, rkt_ref, svr_ref):
        c2e = c2e_ref[...]
        r2e = r2e_ref[...]
        wk0 = wk_ref[0:D, :]
        wk1 = wk_ref[D:2 * D, :]
        wv0 = wv_ref[0:D, :]
        wv1 = wv_ref[D:2 * D, :]
        wo = wo_ref[...]
        scale = 0.25
        sk = jnp.dot(c2e, wk0, preferred_element_type=jnp.float32) * scale
        skt_ref[...] = sk.T
        rk = jnp.dot(r2e, wk1, preferred_element_type=jnp.float32) * scale
        rkt_ref[...] = rk.T
        wvo0 = jnp.dot(wv0, wo, preferred_element_type=jnp.float32)
        wvo1 = jnp.dot(wv1, wo, preferred_element_type=jnp.float32)
        cb = jnp.dot(bv_ref[...], wo, preferred_element_type=jnp.float32) + bo_ref[...]
        sv = jnp.dot(c2e, wvo0, preferred_element_type=jnp.float32)
        rvp = jnp.dot(r2e, wvo1, preferred_element_type=jnp.float32) + cb
        for r in range(NR):
            svr_ref[:, r * D:(r + 1) * D] = sv + rvp[r, :]

    return pl.pallas_call(
        body,
        out_shape=(
            jax.ShapeDtypeStruct((D, NC_PAD), jnp.float32),
            jax.ShapeDtypeStruct((D, NR_PAD), jnp.float32),
            jax.ShapeDtypeStruct((NC_PAD, NR * D), jnp.float32),
        ),
    )(c2e_p, r2e_p, Wk, Wv, Wo, bv2, bo2)


def _sc_vgather(v2eT, nhi, nmod):
    mesh = plsc.VectorSubcoreMesh(core_axis_name="c", subcore_axis_name="s")

    @functools.partial(
        pl.kernel,
        mesh=mesh,
        compiler_params=pltpu.CompilerParams(
            needs_layout_passes=False, use_tc_tiling_on_sc=True),
        out_type=jax.ShapeDtypeStruct((B // 8, 128), jnp.float32),
        scratch_types=[
            pltpu.VMEM((NW, 128), jnp.int32),
            pltpu.VMEM((NW, 128), jnp.int32),
            pltpu.VMEM((16, D, 128), jnp.float32),
            pltpu.VMEM((16, 128), jnp.float32),
            pltpu.SemaphoreType.DMA,
        ],
    )
    def k(v2eT_hbm, nhi_hbm, nmod_hbm, out_hbm, nhi_v, nmod_v, wave, outb, sem):
        wid = lax.axis_index("c") * 16 + lax.axis_index("s")
        pltpu.sync_copy(nhi_hbm, nhi_v)
        pltpu.sync_copy(nmod_hbm, nmod_v)
        iota = lax.iota(jnp.int32, 16)
        for g in range(8):
            hivec = nhi_v[wid, pl.ds(g * 16, 16)]
            modvec = nmod_v[wid, pl.ds(g * 16, 16)]
            copies = []
            for j in range(16):
                c0 = hivec[j] * 128
                copies.append(pltpu.async_copy(
                    v2eT_hbm.at[:, pl.ds(c0, 128)], wave.at[j], sem))
            for cp in copies:
                cp.wait()
            for j in range(16):
                vcrow = plsc.load_gather(
                    wave, [jnp.full((16,), j, jnp.int32), iota,
                           jnp.broadcast_to(modvec[j], (16,))])
                plsc.store_scatter(
                    outb, [jnp.full((16,), 2 * g + j // 8, jnp.int32),
                           (j % 8) * 16 + iota], vcrow)
        pltpu.sync_copy(outb, out_hbm.at[pl.ds(wid * 16, 16)])

    return k(v2eT, nhi, nmod)


def _sc_attention(vcrep, hcomb, Wq, bq2, skt, rkt, svr):
    mesh = plsc.VectorSubcoreMesh(core_axis_name="c", subcore_axis_name="s")
    UNROLL = 2

    @functools.partial(
        pl.kernel,
        mesh=mesh,
        compiler_params=pltpu.CompilerParams(
            needs_layout_passes=False, use_tc_tiling_on_sc=False),
        out_type=jax.ShapeDtypeStruct((B, D), jnp.float32),
        scratch_types=[
            pltpu.VMEM((D, NC_PAD), jnp.float32),
            pltpu.VMEM((D, NR_PAD), jnp.float32),
            pltpu.VMEM((NC_PAD * NR, D), jnp.float32),
            pltpu.VMEM((D, D), jnp.float32),
            pltpu.VMEM((1, D), jnp.float32),
            pltpu.VMEM((ROWS, D), jnp.float32),
            pltpu.VMEM((ROWS, LP), jnp.int32),
            pltpu.VMEM((UNROLL, D), jnp.float32),
            pltpu.VMEM((ROWS, D), jnp.float32),
        ],
    )
    def k(vcrep_hbm, hcomb_hbm, wq_hbm, bq_hbm,
          skt_hbm, rkt_hbm, svr_hbm, out_hbm,
          skt_v, rkt_v, svr_v, wq_v, bq_v, vcrep_v,
          hcomb_v, rkq_buf, outbuf):
        wid = lax.axis_index("c") * 16 + lax.axis_index("s")
        base = wid * ROWS

        pltpu.sync_copy(skt_hbm, skt_v)
        pltpu.sync_copy(rkt_hbm, rkt_v)
        pltpu.sync_copy(svr_hbm, svr_v)
        pltpu.sync_copy(wq_hbm, wq_v)
        pltpu.sync_copy(bq_hbm, bq_v)
        pltpu.sync_copy(vcrep_hbm.at[pl.ds(base, ROWS)], vcrep_v)
        pltpu.sync_copy(hcomb_hbm.at[pl.ds(base, ROWS)], hcomb_v)

        iota = lax.iota(jnp.int32, 16)
        lanemask_last = iota < (L - 3 * 16)
        neg = jnp.full((16,), -1e30, jnp.float32)
        nchunk = LP // 16

        def one_row(i, slot):
            vcvec = vcrep_v[i, :]
            qa = [bq_v[0, :], jnp.zeros((16,), jnp.float32),
                  jnp.zeros((16,), jnp.float32), jnp.zeros((16,), jnp.float32)]
            for d in range(D):
                qa[d % 4] = qa[d % 4] + vcvec[d] * wq_v[d, :]
            q = (qa[0] + qa[1]) + (qa[2] + qa[3])
            qs = [q[d] for d in range(D)]
            ra = [jnp.zeros((16,), jnp.float32) for _ in range(4)]
            for d in range(D):
                ra[d % 4] = ra[d % 4] + qs[d] * rkt_v[d, :]
            rkq_buf[slot, :] = (ra[0] + ra[1]) + (ra[2] + ra[3])
            slotv = jnp.full((16,), slot, jnp.int32)
            chunks, combs = [], []
            for t in range(nchunk):
                packed = hcomb_v[i, pl.ds(16 * t, 16)]
                cv = jnp.bitwise_and(packed, jnp.full((16,), NC_PAD - 1, jnp.int32))
                rr = jnp.right_shift(packed, jnp.full((16,), 10, jnp.int32))
                combs.append(cv * NR + rr)
                sa = [plsc.load_gather(rkq_buf, [slotv, rr]),
                      jnp.zeros((16,), jnp.float32),
                      jnp.zeros((16,), jnp.float32),
                      jnp.zeros((16,), jnp.float32)]
                for d in range(D):
                    dvec = jnp.full((16,), d, jnp.int32)
                    sa[d % 4] = sa[d % 4] + qs[d] * plsc.load_gather(
                        skt_v, [dvec, cv])
                chunks.append((sa[0] + sa[1]) + (sa[2] + sa[3]))
            chunks[3] = jnp.where(lanemask_last, chunks[3], neg)
            m = jnp.max(jnp.maximum(jnp.maximum(chunks[0], chunks[1]),
                                    jnp.maximum(chunks[2], chunks[3])))
            es = [jnp.exp(c - m) for c in chunks]
            total = jnp.sum((es[0] + es[1]) + (es[2] + es[3]))
            inv = jnp.full((16,), 1.0, jnp.float32) / jnp.broadcast_to(
                total, (16,))
            oa = [jnp.zeros((16,), jnp.float32) for _ in range(4)]
            for t in range(nchunk):
                at = es[t] * inv
                for j in range(16):
                    l = 16 * t + j
                    if l >= L:
                        break
                    c = jnp.broadcast_to(combs[t][j], (16,))
                    row = plsc.load_gather(svr_v, [c, iota])
                    oa[l % 4] = oa[l % 4] + at[j] * row
            out = (oa[0] + oa[1]) + (oa[2] + oa[3])
            plsc.store_scatter(outbuf, [jnp.broadcast_to(i, (16,)), iota], out)

        def row_body(ii, carry):
            for u in range(UNROLL):
                one_row(ii * UNROLL + u, u)
            return carry

        lax.fori_loop(0, ROWS // UNROLL, row_body, 0)
        pltpu.sync_copy(outbuf, out_hbm.at[pl.ds(base, ROWS)])

    return k(vcrep, hcomb, Wq, bq2, skt, rkt, svr)


def kernel(nodes, history_vc, history_r, c2e_weight, r2e_weight, v2e_weight,
           Wq, bq, Wk, bk, Wv, bv, Wo, bo):
    nodes = nodes.astype(jnp.int32)
    nhi = (nodes >> 7).reshape(NW, 128)
    nmod = (nodes & 127).reshape(NW, 128)
    v2eT = v2e_weight.T
    vcrep = _sc_vgather(v2eT, nhi, nmod).reshape(B, D)
    hcomb = (history_r.astype(jnp.int32) << 10) | history_vc.astype(jnp.int32)
    hcomb = jnp.pad(hcomb, ((0, 0), (0, LP - L)))
    c2e_p = jnp.pad(c2e_weight, ((0, NC_PAD - c2e_weight.shape[0]), (0, 0)))
    r2e_p = jnp.pad(r2e_weight, ((0, NR_PAD - r2e_weight.shape[0]), (0, 0)))
    bv2 = bv.reshape(1, D)
    bo2 = bo.reshape(1, D)
    bq2 = bq.reshape(1, D)
    skt, rkt, svr80 = _tc_precompute(c2e_p, r2e_p, Wk, Wv, Wo, bv2, bo2)
    svr = svr80.reshape(NC_PAD * NR, D)
    return _sc_attention(vcrep, hcomb, Wq, bq2, skt, rkt, svr)
